# baseline (speedup 1.0000x reference)
import functools

import jax
import jax.numpy as jnp
from jax import lax
from jax.experimental import pallas as pl
from jax.experimental.pallas import tpu as pltpu
from jax.experimental.pallas import tpu_sc as plsc

_M = 4
_V = 100000
_D = 32
_B = 4096
_T = 20
_ROWS = _M * _B * _T
_NW = 32
_NHP = _M * _T * 2
_HPW = _NHP // _NW
_HB = _B // 2
_QB = _HB // 2
_CHUNK = 256
_LANES = 16
_OROWS = _ROWS * _D // 128


def _gather_body(idx_hbm, w_hbm, out_hbm, idx_v, s_v, a_v, gsem, osem):
    c = lax.axis_index("c")
    s = lax.axis_index("s")
    wid = s * 2 + c
    iota = lax.iota(jnp.int32, _LANES)

    def half_plane(i, carry):
        hp = wid * _HPW + i
        plane = hp // 2
        half = hp % 2
        pltpu.sync_copy(idx_hbm.at[pl.ds(hp * _HB, _HB)], idx_v)

        off = (hp // (_T * 2)) * _V

        def add_off(k, cc):
            for u in range(4):
                sl = pl.ds((k * 4 + u) * _LANES, _LANES)
                idx_v[sl] = idx_v[sl] + off
            return cc

        lax.fori_loop(0, _HB // (4 * _LANES), add_off, 0)

        for q in range(2):
            for j in range(_QB // _CHUNK):
                pltpu.async_copy(
                    w_hbm.at[idx_v.at[pl.ds(q * _QB + j * _CHUNK, _CHUNK)]],
                    s_v.at[pl.ds(j * _CHUNK, _CHUNK)], gsem)
            pltpu.make_async_copy(
                w_hbm.at[pl.ds(0, _QB)], s_v, gsem).wait()

            def transpose_group(g, cc):
                rowv = iota + g * _LANES
                btlq = g // 8
                acol = (g % 8) * _LANES
                for d in range(_D):
                    vals = plsc.load_gather(
                        s_v, [rowv, jnp.full((_LANES,), d, jnp.int32)])
                    arow = (d // 8) * 64 + (d % 8) + btlq * 8
                    a_v[arow, pl.ds(acol, _LANES)] = vals
                return cc

            lax.fori_loop(0, _QB // _LANES, transpose_group, 0)

            for dt in range(4):
                orow = ((plane * 4 + dt) * 32 + half * 16 + q * 8) * 8
                pltpu.async_copy(
                    a_v.at[pl.ds(dt * 64, 64)],
                    out_hbm.at[pl.ds(orow, 64)], osem)
            for dt in range(4):
                pltpu.make_async_copy(
                    a_v.at[pl.ds(dt * 64, 64)],
                    out_hbm.at[pl.ds(0, 64)], osem).wait()
        return carry

    lax.fori_loop(0, _HPW, half_plane, 0)


@jax.jit
def _run(idx_t, w_flat):
    mesh = plsc.VectorSubcoreMesh(core_axis_name="c", subcore_axis_name="s")
    f = functools.partial(
        pl.kernel,
        mesh=mesh,
        out_type=jax.ShapeDtypeStruct((_OROWS, 128), jnp.float32),
        scratch_types=[
            pltpu.VMEM((_HB,), jnp.int32),
            pltpu.VMEM((_QB, _D), jnp.float32),
            pltpu.VMEM((256, 128), jnp.float32),
            pltpu.SemaphoreType.DMA,
            pltpu.SemaphoreType.DMA,
        ],
        compiler_params=pltpu.CompilerParams(
            use_tc_tiling_on_sc=False, needs_layout_passes=False),
    )(_gather_body)
    return f(idx_t, w_flat)


def kernel(idx, weight):
    idx_t = jnp.transpose(idx, (0, 2, 1)).reshape(_ROWS).astype(jnp.int32)
    w_flat = weight.reshape(_M * _V, _D)
    out = _run(idx_t, w_flat)
    o6 = out.reshape(_M, _T, 4, 32, 8, 128)
    return o6.transpose(0, 3, 5, 1, 2, 4).reshape(_M, _B, _T, _D)


# speedup vs baseline: 1.2067x; 1.2067x over previous
import functools

import jax
import jax.numpy as jnp
from jax import lax
from jax.experimental import pallas as pl
from jax.experimental.pallas import tpu as pltpu
from jax.experimental.pallas import tpu_sc as plsc

_M = 4
_V = 100000
_D = 32
_B = 4096
_T = 20
_ROWS = _M * _B * _T
_NW = 32
_RPW = _ROWS // _NW
_CHUNK = 256
_CPR = 5
_RROWS = _CPR * _CHUNK
_NR = _RPW // _RROWS
_NBUF = 2
_LANES = 16
_VC = 1000


def _gather_body(idx_hbm, w_hbm, out_hbm, idx_v, buf0, buf1,
                 gsem0, gsem1, osem0, osem1):
    c = lax.axis_index("c")
    s = lax.axis_index("s")
    wid = s * 2 + c
    base = wid * _RPW
    pltpu.sync_copy(idx_hbm.at[pl.ds(base, _RPW)], idx_v)

    off = (base // (_B * _T)) * _V

    def add_off(i, carry):
        for u in range(4):
            sl = pl.ds((i * 4 + u) * _LANES, _LANES)
            idx_v[sl] = idx_v[sl] + off
        return carry

    lax.fori_loop(0, _RPW // (4 * _LANES), add_off, 0)

    def fire_gathers(r, buf, gsem):
        for j in range(_CPR):
            k = r * _RROWS + j * _CHUNK
            pltpu.async_copy(
                w_hbm.at[idx_v.at[pl.ds(k, _CHUNK)]],
                buf.at[pl.ds(j * _CHUNK, _CHUNK)], gsem)

    def drain(buf, sem):
        pltpu.make_async_copy(
            out_hbm.at[pl.ds(0, _RROWS)], buf, sem).wait()

    fire_gathers(0, buf0, gsem0)
    fire_gathers(1, buf1, gsem1)

    def body(i, carry):
        for half, buf, gsem, osem in (
                (0, buf0, gsem0, osem0), (1, buf1, gsem1, osem1)):
            r = i * _NBUF + half
            drain(buf, gsem)
            pltpu.async_copy(
                buf, out_hbm.at[pl.ds(base + r * _RROWS, _RROWS)], osem)
            drain(buf, osem)

            @pl.when(r < _NR - _NBUF)
            def _():
                fire_gathers(r + _NBUF, buf, gsem)
        return carry

    lax.fori_loop(0, _NR // _NBUF, body, 0)


@jax.jit
def _run(idx_t, w_v):
    eye = jnp.eye(_D, dtype=jnp.float32)
    w_rm = jnp.einsum('mdv,de->mve', w_v, eye)
    w_flat = w_rm.reshape(_M * _V, _D)

    mesh = plsc.VectorSubcoreMesh(core_axis_name="c", subcore_axis_name="s")
    f = functools.partial(
        pl.kernel,
        mesh=mesh,
        out_type=jax.ShapeDtypeStruct((_ROWS, _D), jnp.float32),
        scratch_types=[
            pltpu.VMEM((_RPW,), jnp.int32),
            pltpu.VMEM((_RROWS, _D), jnp.float32),
            pltpu.VMEM((_RROWS, _D), jnp.float32),
            pltpu.SemaphoreType.DMA,
            pltpu.SemaphoreType.DMA,
            pltpu.SemaphoreType.DMA,
            pltpu.SemaphoreType.DMA,
        ],
        compiler_params=pltpu.CompilerParams(use_tc_tiling_on_sc=False),
    )(_gather_body)
    out = f(idx_t, w_flat)

    o5 = out.reshape(_M * _T, _B, _D)
    o_t = jnp.einsum('dc,pbc->pdb', eye, o5)
    return o_t


def kernel(idx, weight):
    idx_t = jnp.transpose(idx, (0, 2, 1)).reshape(_ROWS).astype(jnp.int32)
    w_v = jnp.swapaxes(weight, 1, 2)
    o_t = _run(idx_t, w_v)
    o4 = o_t.reshape(_M, _T, _D, _B)
    return jnp.transpose(o4, (0, 3, 1, 2))


# speedup vs baseline: 1.2996x; 1.0770x over previous
import functools

import jax
import jax.numpy as jnp
from jax import lax
from jax.experimental import pallas as pl
from jax.experimental.pallas import tpu as pltpu
from jax.experimental.pallas import tpu_sc as plsc

_M = 4
_V = 100000
_D = 32
_B = 4096
_T = 20
_ROWS = _M * _B * _T
_NW = 32
_RPW = _ROWS // _NW
_CHUNK = 256
_CPR = 5
_RROWS = _CPR * _CHUNK
_NR = _RPW // _RROWS
_NBUF = 2
_LANES = 16


def _gather_body(idx_hbm, w_hbm, out_hbm, idx_v, buf0, buf1,
                 gsem0, gsem1, osem0, osem1):
    c = lax.axis_index("c")
    s = lax.axis_index("s")
    wid = s * 2 + c
    base = wid * _RPW
    pltpu.sync_copy(idx_hbm.at[pl.ds(base, _RPW)], idx_v)

    off = (base // (_B * _T)) * _V

    def add_off(i, carry):
        for u in range(4):
            sl = pl.ds((i * 4 + u) * _LANES, _LANES)
            idx_v[sl] = idx_v[sl] + off
        return carry

    lax.fori_loop(0, _RPW // (4 * _LANES), add_off, 0)

    def fire_gathers(r, buf, gsem):
        for j in range(_CPR):
            k = r * _RROWS + j * _CHUNK
            pltpu.async_copy(
                w_hbm.at[idx_v.at[pl.ds(k, _CHUNK)]],
                buf.at[pl.ds(j * _CHUNK, _CHUNK)], gsem)

    def drain(buf, sem):
        pltpu.make_async_copy(
            out_hbm.at[pl.ds(0, _RROWS)], buf, sem).wait()

    fire_gathers(0, buf0, gsem0)
    fire_gathers(1, buf1, gsem1)

    def body(i, carry):
        for half, buf, gsem, osem in (
                (0, buf0, gsem0, osem0), (1, buf1, gsem1, osem1)):
            r = i * _NBUF + half
            drain(buf, gsem)
            pltpu.async_copy(
                buf, out_hbm.at[pl.ds(base + r * _RROWS, _RROWS)], osem)
            drain(buf, osem)

            @pl.when(r < _NR - _NBUF)
            def _():
                fire_gathers(r + _NBUF, buf, gsem)
        return carry

    lax.fori_loop(0, _NR // _NBUF, body, 0)


@jax.jit
def _run(idx_t, w_flat):

    mesh = plsc.VectorSubcoreMesh(core_axis_name="c", subcore_axis_name="s")
    f = functools.partial(
        pl.kernel,
        mesh=mesh,
        out_type=jax.ShapeDtypeStruct((_ROWS, _D), jnp.float32),
        scratch_types=[
            pltpu.VMEM((_RPW,), jnp.int32),
            pltpu.VMEM((_RROWS, _D), jnp.float32),
            pltpu.VMEM((_RROWS, _D), jnp.float32),
            pltpu.SemaphoreType.DMA,
            pltpu.SemaphoreType.DMA,
            pltpu.SemaphoreType.DMA,
            pltpu.SemaphoreType.DMA,
        ],
        compiler_params=pltpu.CompilerParams(use_tc_tiling_on_sc=False),
    )(_gather_body)
    out = f(idx_t, w_flat)

    eye = jnp.eye(_D, dtype=jnp.float32)
    o5 = out.reshape(_M * _T, _B, _D)
    o_t = jnp.einsum('dc,pbc->pdb', eye, o5)
    return o_t


def kernel(idx, weight):
    idx_t = jnp.transpose(idx, (0, 2, 1)).reshape(_ROWS).astype(jnp.int32)
    w_flat = weight.reshape(_M * _V, _D)
    o_t = _run(idx_t, w_flat)
    o4 = o_t.reshape(_M, _T, _D, _B)
    return jnp.transpose(o4, (0, 3, 1, 2))
